# Initial kernel scaffold; baseline (speedup 1.0000x reference)
import jax
import jax.numpy as jnp
from jax.experimental import pallas as pl


def kernel(x, table):
    raise NotImplementedError("write your pallas kernel here")


# speedup vs baseline: 3.5691x; 3.5691x over previous
import functools

import jax
import jax.numpy as jnp
from jax import lax
from jax.experimental import pallas as pl
from jax.experimental.pallas import tpu as pltpu
from jax.experimental.pallas import tpu_sc as plsc

NUM_ROWS = 1000
DIM = 64
BATCH = 4096 * 200

NC = 2
NS = 16
NW = NC * NS
B_PER_W = BATCH // NW
CHUNK = 128
N_CHUNKS = B_PER_W // CHUNK


@functools.partial(
    pl.kernel,
    out_type=jax.ShapeDtypeStruct((BATCH, DIM), jnp.float32),
    mesh=plsc.VectorSubcoreMesh(core_axis_name="c", subcore_axis_name="s"),
    scratch_types=[
        pltpu.VMEM((B_PER_W,), jnp.int32),
        pltpu.VMEM((CHUNK, DIM), jnp.float32),
        pltpu.VMEM((CHUNK, DIM), jnp.float32),
        pltpu.SemaphoreType.DMA,
        pltpu.SemaphoreType.DMA,
    ],
    compiler_params=pltpu.CompilerParams(use_tc_tiling_on_sc=False),
)
def _lookup(x_hbm, table_hbm, out_hbm, idx_v, buf0, buf1, sem0, sem1):
    wid = lax.axis_index("s") * NC + lax.axis_index("c")
    base = wid * B_PER_W

    pltpu.sync_copy(x_hbm.at[pl.ds(base, B_PER_W)], idx_v)

    def issue(i, buf, sem):
        src = table_hbm.at[idx_v.at[pl.ds(i * CHUNK, CHUNK)]]
        pltpu.make_async_copy(src, buf, sem).start()

    def drain(buf, sem):
        src = table_hbm.at[idx_v.at[pl.ds(0, CHUNK)]]
        pltpu.make_async_copy(src, buf, sem).wait()

    def out_copy(i, buf):
        pltpu.sync_copy(buf, out_hbm.at[pl.ds(base + i * CHUNK, CHUNK)])

    issue(0, buf0, sem0)

    @pl.loop(0, N_CHUNKS, step=2)
    def _(t):
        issue(t + 1, buf1, sem1)
        drain(buf0, sem0)
        out_copy(t, buf0)

        @pl.when(t + 2 < N_CHUNKS)
        def _():
            issue(t + 2, buf0, sem0)

        drain(buf1, sem1)
        out_copy(t + 1, buf1)


def kernel(x, table):
    out = _lookup(x.reshape(-1), table)
    return out.reshape(x.shape[0], x.shape[1], DIM)
